# Initial kernel scaffold; baseline (speedup 1.0000x reference)
import jax
import jax.numpy as jnp
from jax.experimental import pallas as pl


def kernel(Theta, W0, Wc, bc, W1, b1, W2, b2, W3, b3):
    raise NotImplementedError("write your pallas kernel here")


# speedup vs baseline: 15.1111x; 15.1111x over previous
import functools
import math

import jax
import jax.numpy as jnp
from jax import lax
from jax.experimental import pallas as pl
from jax.experimental.pallas import tpu as pltpu
from jax.experimental.pallas import tpu_sc as plsc

P = 16
_SQRT_P = math.sqrt(P)


def _bcast_lane(scr, idx):
    return plsc.load_gather(scr, [idx])


def _spodnet_body(theta_hbm, wpack_hbm, out_hbm, Th, Wp, sA, sB):
    c = lax.axis_index("c")
    s = lax.axis_index("s")

    @pl.when(jnp.logical_and(c == 0, s == 0))
    def _():
        pltpu.sync_copy(theta_hbm, Th)
        pltpu.sync_copy(wpack_hbm, Wp)

        iot = lax.iota(jnp.int32, P)
        zero = jnp.zeros((P,), jnp.float32)
        lane0 = iot == 0

        def cj(j):
            return jnp.broadcast_to(jnp.int32(j), (P,))

        def pass1(col, _):
            colv = jnp.broadcast_to(col, (P,))
            t = plsc.load_gather(Th, [iot, colv])
            sA[...] = t
            idx12 = jnp.where(iot < colv, iot, jnp.minimum(iot + 1, P - 1))
            t12 = jnp.where(iot < P - 1, plsc.load_gather(sA, [idx12]), 0.0)
            sB[...] = t12
            acc = Wp[64, :]
            for j in range(P - 1):
                acc = acc + _bcast_lane(sB, cj(j)) * Wp[16 + j, :]
            y = acc * jnp.float32(1.0 / _SQRT_P)
            diff15 = y - t12
            sA[...] = diff15
            inv = iot - jnp.where(iot > colv, 1, 0)
            dfull = jnp.where(iot == colv, 0.0, plsc.load_gather(sA, [inv]))
            plsc.addupdate_scatter(Th, [iot, colv], dfull)
            plsc.addupdate_scatter(Th, [colv, iot], dfull)
            return 0

        lax.fori_loop(0, P, pass1, 0)

        def pass2(col, _):
            colv = jnp.broadcast_to(col, (P,))
            colmask = iot == colv
            t = plsc.load_gather(Th, [iot, colv])
            t22 = plsc.load_gather(Th, [colv, colv])
            u = jnp.where(colmask, 0.0, t)
            w22 = plsc.load_gather(Wp, [colv, colv])
            wcol = plsc.load_gather(Wp, [iot, colv])
            v = jnp.where(colmask, 0.0, wcol)
            winv = 1.0 / w22
            sB[...] = v
            a = []
            for i in range(P):
                vi = _bcast_lane(sB, cj(i))
                a.append(Wp[i, :] - winv * (vi * v))
            sA[...] = u
            m = zero
            for j in range(P):
                m = m + _bcast_lane(sA, cj(j)) * a[j]
            m = jnp.where(colmask, 0.0, m)
            schur = jnp.sum(u * m)
            sA[...] = t
            perm = jnp.where(iot == 0, colv,
                             jnp.where(iot <= colv, iot - 1, iot))
            feats = plsc.load_gather(sA, [perm])
            sA[...] = feats
            h = Wp[65, :]
            for j in range(P):
                h = h + _bcast_lane(sA, cj(j)) * Wp[32 + j, :]
            h = jnp.maximum(h, 0.0)
            sA[...] = h
            h2 = Wp[66, :]
            for j in range(P):
                h2 = h2 + _bcast_lane(sA, cj(j)) * Wp[48 + j, :]
            h2 = jnp.maximum(h2, 0.0)
            gy = jnp.exp(jnp.broadcast_to(jnp.sum(h2 * Wp[67, :]), (P,))
                         + Wp[68, :])
            diag = t22 + ((gy + schur) - t22)
            plsc.store_scatter(Th, [colv, colv], diag, mask=lane0)
            w22n = 1.0 / gy
            w12n = (-w22n) * m
            sA[...] = w12n
            rowc = jnp.where(colmask, w22n, w12n)
            for i in range(P):
                wni = _bcast_lane(sA, cj(i))
                g = a[i] + gy * (wni * w12n)
                row = jnp.where(colmask, wni, g)
                Wp[i, :] = jnp.where(colv == i, rowc, row)
            return 0

        lax.fori_loop(0, P, pass2, 0)

        pltpu.sync_copy(Th, out_hbm)


@functools.lru_cache(maxsize=None)
def _spodnet_sc():
    mesh = plsc.VectorSubcoreMesh(core_axis_name="c", subcore_axis_name="s")
    return pl.kernel(
        _spodnet_body,
        out_type=jax.ShapeDtypeStruct((P, P), jnp.float32),
        mesh=mesh,
        compiler_params=pltpu.CompilerParams(needs_layout_passes=False),
        scratch_types=[
            pltpu.VMEM((P, P), jnp.float32),
            pltpu.VMEM((72, P), jnp.float32),
            pltpu.VMEM((P,), jnp.float32),
            pltpu.VMEM((P,), jnp.float32),
        ],
    )


def kernel(Theta, W0, Wc, bc, W1, b1, W2, b2, W3, b3):
    f32 = jnp.float32
    theta2d = Theta[0].astype(f32)
    wct = jnp.zeros((P, P), f32).at[: P - 1, : P - 1].set(Wc.T.astype(f32))
    bcp = jnp.zeros((P,), f32).at[: P - 1].set(bc.astype(f32))
    wpack = jnp.concatenate(
        [
            W0[0].astype(f32),
            wct,
            W1.T.astype(f32),
            W2.T.astype(f32),
            bcp[None, :],
            b1.astype(f32)[None, :],
            b2.astype(f32)[None, :],
            W3[0].astype(f32)[None, :],
            jnp.broadcast_to(b3.astype(f32)[0], (P,))[None, :],
            jnp.zeros((3, P), f32),
        ],
        axis=0,
    )
    out = _spodnet_sc()(theta2d, wpack)
    return out[None, :, :]
